# baseline (speedup 1.0000x reference)
import functools

import jax
import jax.numpy as jnp
from jax.experimental import pallas as pl

_N = 10000
_E = 160000
_F = 26
_H = 128
_T = 20
_L = 2
_NM = 2500
_NL = 625
_NMP = 2560
_NLP = 640
_NP = 10240


def _leaky(v):
    return jnp.where(v > 0, v, 0.01 * v)


def _enc_body(x_ref, w1_ref, b1_ref, w2_ref, b2_ref, ws_ref, o_ref):
    xt = x_ref[0]
    h = _leaky(jnp.dot(xt, w1_ref[...], preferred_element_type=jnp.float32)
               + b1_ref[...])
    o_ref[0] = (jnp.dot(h, w2_ref[...], preferred_element_type=jnp.float32)
                + b2_ref[...]
                + jnp.dot(xt, ws_ref[...], preferred_element_type=jnp.float32))


def _encode(xT, W1, b1, W2, b2, Ws):
    return pl.pallas_call(
        _enc_body,
        grid=(_T,),
        in_specs=[
            pl.BlockSpec((1, _N, _F), lambda t: (t, 0, 0)),
            pl.BlockSpec((_F, _H), lambda t: (0, 0)),
            pl.BlockSpec((1, _H), lambda t: (0, 0)),
            pl.BlockSpec((_H, _H), lambda t: (0, 0)),
            pl.BlockSpec((1, _H), lambda t: (0, 0)),
            pl.BlockSpec((_F, _H), lambda t: (0, 0)),
        ],
        out_specs=pl.BlockSpec((1, _N, _H), lambda t: (t, 0, 0)),
        out_shape=jax.ShapeDtypeStruct((_T, _N, _H), jnp.float32),
    )(xT, W1, b1[None, :], W2, b2[None, :], Ws)


def _coarsen_mid_body(x_ref, wn_ref, bn_ref, o_ref):
    xt = x_ref[0]
    h = _leaky(jnp.dot(xt, wn_ref[...], preferred_element_type=jnp.float32)
               + bn_ref[...])
    r = jax.lax.broadcasted_iota(jnp.int32, (512, 2048), 0)
    c = jax.lax.broadcasted_iota(jnp.int32, (512, 2048), 1)
    pool = jnp.where(c // 4 == r, 0.25, 0.0)
    o_ref[0] = jnp.dot(pool, h, preferred_element_type=jnp.float32)


def _coarsen_mid(xTp, Wn, bn):
    return pl.pallas_call(
        _coarsen_mid_body,
        grid=(_T, 5),
        in_specs=[
            pl.BlockSpec((1, 2048, _F), lambda t, i: (t, i, 0)),
            pl.BlockSpec((_F, _H), lambda t, i: (0, 0)),
            pl.BlockSpec((1, _H), lambda t, i: (0, 0)),
        ],
        out_specs=pl.BlockSpec((1, 512, _H), lambda t, i: (t, i, 0)),
        out_shape=jax.ShapeDtypeStruct((_T, _NMP, _H), jnp.float32),
    )(xTp, Wn, bn[None, :])


def _coarsen_large_body(x_ref, wn_ref, bn_ref, o_ref):
    xt = x_ref[0]
    h = _leaky(jnp.dot(xt, wn_ref[...], preferred_element_type=jnp.float32)
               + bn_ref[...])
    o_ref[0] = h.reshape(_NLP, 16, _H).sum(axis=1) * (1.0 / 16.0)


def _coarsen_large(xTp, Wn, bn):
    return pl.pallas_call(
        _coarsen_large_body,
        grid=(_T,),
        in_specs=[
            pl.BlockSpec((1, _NP, _F), lambda t: (t, 0, 0)),
            pl.BlockSpec((_F, _H), lambda t: (0, 0)),
            pl.BlockSpec((1, _H), lambda t: (0, 0)),
        ],
        out_specs=pl.BlockSpec((1, _NLP, _H), lambda t: (t, 0, 0)),
        out_shape=jax.ShapeDtypeStruct((_T, _NLP, _H), jnp.float32),
    )(xTp, Wn, bn[None, :])


def _gru_body(full_out, agg_ref, xseq_ref, norm_ref, hprev_ref, wx_ref,
              whzr_ref, wh2_ref, b_ref, a_ref, out_ref, hlast_ref):
    Hst0 = hprev_ref[...]

    def step(t, Hst):
        aggt = agg_ref[t]
        G = (norm_ref[...]
             * jnp.dot(aggt, wx_ref[...], preferred_element_type=jnp.float32)
             + b_ref[...])
        Hw = jnp.dot(Hst, whzr_ref[...], preferred_element_type=jnp.float32)
        z = jax.nn.sigmoid(G[:, :_H] + Hw[:, :_H])
        r = jax.nn.sigmoid(G[:, _H:2 * _H] + Hw[:, _H:])
        htil = jnp.tanh(G[:, 2 * _H:]
                        + jnp.dot(r * Hst, wh2_ref[...],
                                  preferred_element_type=jnp.float32))
        Hst = z * Hst + (1.0 - z) * htil
        if full_out:
            out_ref[t] = a_ref[t] * Hst + xseq_ref[t]
        return Hst

    Hst = jax.lax.fori_loop(0, _T, step, Hst0)
    hl = a_ref[_T - 1] * Hst
    hlast_ref[...] = hl
    if not full_out:
        out_ref[...] = hl + xseq_ref[_T - 1]


def _a3tgcn_layer(agg, xseq, norm, hprev, Wx, Wh, b, att, n, blk, full_out):
    nb = n // blk
    wx_cat = jnp.concatenate([Wx[0], Wx[1], Wx[2]], axis=1)
    whzr = jnp.concatenate([Wh[0], Wh[1]], axis=1)
    b_cat = jnp.concatenate([b[0], b[1], b[2]])[None, :]
    a = jax.nn.softmax(att)
    a3 = jnp.broadcast_to(a[:, None, None], (_T, 1, _H))
    if full_out:
        out_shape = jax.ShapeDtypeStruct((_T, n, _H), jnp.float32)
        out_spec = pl.BlockSpec((_T, blk, _H), lambda i: (0, i, 0))
    else:
        out_shape = jax.ShapeDtypeStruct((n, _H), jnp.float32)
        out_spec = pl.BlockSpec((blk, _H), lambda i: (i, 0))
    out, hlast = pl.pallas_call(
        functools.partial(_gru_body, full_out),
        grid=(nb,),
        in_specs=[
            pl.BlockSpec((_T, blk, _H), lambda i: (0, i, 0)),
            pl.BlockSpec((_T, blk, _H), lambda i: (0, i, 0)),
            pl.BlockSpec((blk, 1), lambda i: (i, 0)),
            pl.BlockSpec((blk, _H), lambda i: (i, 0)),
            pl.BlockSpec((_H, 3 * _H), lambda i: (0, 0)),
            pl.BlockSpec((_H, 2 * _H), lambda i: (0, 0)),
            pl.BlockSpec((_H, _H), lambda i: (0, 0)),
            pl.BlockSpec((1, 3 * _H), lambda i: (0, 0)),
            pl.BlockSpec((_T, 1, _H), lambda i: (0, 0, 0)),
        ],
        out_specs=[out_spec, pl.BlockSpec((blk, _H), lambda i: (i, 0))],
        out_shape=[out_shape, jax.ShapeDtypeStruct((n, _H), jnp.float32)],
    )(agg, xseq, norm, hprev, wx_cat, whzr, Wh[2], b_cat, a3)
    return out, hlast


def _dec_body(v_ref, w_ref, b_ref, o_ref):
    o_ref[...] = _leaky(jnp.dot(v_ref[...], w_ref[...],
                                preferred_element_type=jnp.float32)
                        + b_ref[...])


def _decode(v, W, b):
    n = v.shape[0]
    return pl.pallas_call(
        _dec_body,
        in_specs=[pl.BlockSpec((n, _H), lambda: (0, 0)),
                  pl.BlockSpec((_H, _H), lambda: (0, 0)),
                  pl.BlockSpec((1, _H), lambda: (0, 0))],
        out_specs=pl.BlockSpec((n, _H), lambda: (0, 0)),
        out_shape=jax.ShapeDtypeStruct((n, _H), jnp.float32),
    )(v, W, b[None, :])


def _readout_body(ef_ref, m_ref, l_ref, xl_ref, w1_ref, b1_ref, w2_ref,
                  b2_ref, ws_ref, o_ref):
    xs = jnp.concatenate([ef_ref[...], m_ref[...], l_ref[...]], axis=1)
    h = _leaky(jnp.dot(xs, w1_ref[...], preferred_element_type=jnp.float32)
               + b1_ref[...])
    o_ref[...] = (jnp.dot(h, w2_ref[...], preferred_element_type=jnp.float32)
                  + b2_ref[...]
                  + jnp.dot(xs, ws_ref[...], preferred_element_type=jnp.float32)
                  + 0.3 * xl_ref[...])


def _readout(ef, m2f, l2f, x_last, W1, b1, W2, b2, Ws):
    blk = 2000
    return pl.pallas_call(
        _readout_body,
        grid=(_N // blk,),
        in_specs=[
            pl.BlockSpec((blk, _H), lambda i: (i, 0)),
            pl.BlockSpec((blk, _H), lambda i: (i, 0)),
            pl.BlockSpec((blk, _H), lambda i: (i, 0)),
            pl.BlockSpec((blk, _F), lambda i: (i, 0)),
            pl.BlockSpec((3 * _H, _H), lambda i: (0, 0)),
            pl.BlockSpec((1, _H), lambda i: (0, 0)),
            pl.BlockSpec((_H, _F), lambda i: (0, 0)),
            pl.BlockSpec((1, _F), lambda i: (0, 0)),
            pl.BlockSpec((3 * _H, _F), lambda i: (0, 0)),
        ],
        out_specs=pl.BlockSpec((blk, _F), lambda i: (i, 0)),
        out_shape=jax.ShapeDtypeStruct((_N, _F), jnp.float32),
    )(ef, m2f, l2f, x_last, W1, b1[None, :], W2, b2[None, :], Ws)


def _agg_T(xseq, src, dst, n):
    def one(xt):
        return jax.ops.segment_sum(jnp.take(xt, src, axis=0), dst,
                                   num_segments=n)
    return jax.lax.map(one, xseq)


def _deg_norm(dst, n):
    deg = jax.ops.segment_sum(jnp.ones_like(dst, dtype=jnp.float32), dst,
                              num_segments=n)
    return (1.0 / jnp.clip(deg, 1.0))[:, None]


def kernel(x, edge_index, enc_W1, enc_b1, enc_W2, enc_b2, enc_Ws, cm_Wn,
           cm_bn, cm_We, cm_be, cl_Wn, cl_bn, cl_We, cl_be, gru_Wx, gru_Wh,
           gru_b, att, dm_W, dm_b, dl_W, dl_b, ro_W1, ro_b1, ro_W2, ro_b2,
           ro_Ws):
    src, dst = edge_index[0], edge_index[1]
    msrc, mdst = src[::4] // 4, dst[::4] // 4
    lsrc, ldst = src[::16] // 16, dst[::16] // 16

    norm_f = _deg_norm(dst, _N)
    norm_m = _deg_norm(mdst, _NMP)
    norm_l = _deg_norm(ldst, _NLP)

    xT = jnp.transpose(x, (1, 0, 2))
    xTp = jnp.concatenate(
        [xT, jnp.zeros((_T, _NP - _N, _F), jnp.float32)], axis=1)
    enc = _encode(xT, enc_W1, enc_b1, enc_W2, enc_b2, enc_Ws)
    mid = _coarsen_mid(xTp, cm_Wn, cm_bn)
    large = _coarsen_large(xTp, cl_Wn, cl_bn)

    hf = jnp.zeros((_N, _H), jnp.float32)
    hm = jnp.zeros((_NMP, _H), jnp.float32)
    hl = jnp.zeros((_NLP, _H), jnp.float32)

    for layer in range(_L):
        full = layer < _L - 1
        agg_f = _agg_T(enc, src, dst, _N)
        agg_m = _agg_T(mid, msrc, mdst, _NMP)
        agg_l = _agg_T(large, lsrc, ldst, _NLP)
        enc, hf = _a3tgcn_layer(agg_f, enc, norm_f, hf, gru_Wx[0, layer],
                                gru_Wh[0, layer], gru_b[0, layer],
                                att[0, layer], _N, 400, full)
        mid, hm = _a3tgcn_layer(agg_m, mid, norm_m, hm, gru_Wx[1, layer],
                                gru_Wh[1, layer], gru_b[1, layer],
                                att[1, layer], _NMP, 512, full)
        large, hl = _a3tgcn_layer(agg_l, large, norm_l, hl, gru_Wx[2, layer],
                                  gru_Wh[2, layer], gru_b[2, layer],
                                  att[2, layer], _NLP, _NLP, full)

    m2f = _decode(mid, dm_W, dm_b)[:_NM]
    l2f = _decode(large, dl_W, dl_b)[:_NL]
    m2f = jnp.broadcast_to(m2f[:, None, :], (_NM, 4, _H)).reshape(_N, _H)
    l2f = jnp.broadcast_to(l2f[:, None, :], (_NL, 16, _H)).reshape(_N, _H)

    out = _readout(enc, m2f, l2f, x[:, -1, :], ro_W1, ro_b1, ro_W2, ro_b2,
                   ro_Ws)
    return out[:, None, :]


# speedup vs baseline: 1.7448x; 1.7448x over previous
import functools

import jax
import jax.numpy as jnp
from jax.experimental import pallas as pl

_N = 10000
_E = 160000
_F = 26
_H = 128
_T = 20
_L = 2
_NM = 2500
_NL = 625
_NMP = 2560
_NLP = 640
_NP = 10240


def _leaky(v):
    return jnp.where(v > 0, v, 0.01 * v)


def _enc_body(x_ref, w1_ref, b1_ref, w2_ref, b2_ref, ws_ref, o_ref):
    xt = x_ref[0]
    h = _leaky(jnp.dot(xt, w1_ref[...], preferred_element_type=jnp.float32)
               + b1_ref[...])
    o_ref[0] = (jnp.dot(h, w2_ref[...], preferred_element_type=jnp.float32)
                + b2_ref[...]
                + jnp.dot(xt, ws_ref[...], preferred_element_type=jnp.float32))


def _encode(xT, W1, b1, W2, b2, Ws):
    return pl.pallas_call(
        _enc_body,
        grid=(_T,),
        in_specs=[
            pl.BlockSpec((1, _N, _F), lambda t: (t, 0, 0)),
            pl.BlockSpec((_F, _H), lambda t: (0, 0)),
            pl.BlockSpec((1, _H), lambda t: (0, 0)),
            pl.BlockSpec((_H, _H), lambda t: (0, 0)),
            pl.BlockSpec((1, _H), lambda t: (0, 0)),
            pl.BlockSpec((_F, _H), lambda t: (0, 0)),
        ],
        out_specs=pl.BlockSpec((1, _N, _H), lambda t: (t, 0, 0)),
        out_shape=jax.ShapeDtypeStruct((_T, _N, _H), jnp.float32),
    )(xT, W1, b1[None, :], W2, b2[None, :], Ws)


def _coarsen_mid_body(x_ref, wn_ref, bn_ref, o_ref):
    xt = x_ref[0]
    h = _leaky(jnp.dot(xt, wn_ref[...], preferred_element_type=jnp.float32)
               + bn_ref[...])
    r = jax.lax.broadcasted_iota(jnp.int32, (512, 2048), 0)
    c = jax.lax.broadcasted_iota(jnp.int32, (512, 2048), 1)
    pool = jnp.where(c // 4 == r, 0.25, 0.0)
    o_ref[0] = jnp.dot(pool, h, preferred_element_type=jnp.float32)


def _coarsen_mid(xTp, Wn, bn):
    return pl.pallas_call(
        _coarsen_mid_body,
        grid=(_T, 5),
        in_specs=[
            pl.BlockSpec((1, 2048, _F), lambda t, i: (t, i, 0)),
            pl.BlockSpec((_F, _H), lambda t, i: (0, 0)),
            pl.BlockSpec((1, _H), lambda t, i: (0, 0)),
        ],
        out_specs=pl.BlockSpec((1, 512, _H), lambda t, i: (t, i, 0)),
        out_shape=jax.ShapeDtypeStruct((_T, _NMP, _H), jnp.float32),
    )(xTp, Wn, bn[None, :])


def _coarsen_large_body(x_ref, wn_ref, bn_ref, o_ref):
    xt = x_ref[0]
    h = _leaky(jnp.dot(xt, wn_ref[...], preferred_element_type=jnp.float32)
               + bn_ref[...])
    o_ref[0] = h.reshape(_NLP, 16, _H).sum(axis=1) * (1.0 / 16.0)


def _coarsen_large(xTp, Wn, bn):
    return pl.pallas_call(
        _coarsen_large_body,
        grid=(_T,),
        in_specs=[
            pl.BlockSpec((1, _NP, _F), lambda t: (t, 0, 0)),
            pl.BlockSpec((_F, _H), lambda t: (0, 0)),
            pl.BlockSpec((1, _H), lambda t: (0, 0)),
        ],
        out_specs=pl.BlockSpec((1, _NLP, _H), lambda t: (t, 0, 0)),
        out_shape=jax.ShapeDtypeStruct((_T, _NLP, _H), jnp.float32),
    )(xTp, Wn, bn[None, :])


def _gru_body(full_out, agg_ref, xseq_ref, norm_ref, hprev_ref, wx_ref,
              whzr_ref, wh2_ref, b_ref, a_ref, out_ref, hlast_ref):
    Hst0 = hprev_ref[...]

    def step(t, Hst):
        aggt = agg_ref[t]
        G = (norm_ref[...]
             * jnp.dot(aggt, wx_ref[...], preferred_element_type=jnp.float32)
             + b_ref[...])
        Hw = jnp.dot(Hst, whzr_ref[...], preferred_element_type=jnp.float32)
        z = jax.nn.sigmoid(G[:, :_H] + Hw[:, :_H])
        r = jax.nn.sigmoid(G[:, _H:2 * _H] + Hw[:, _H:])
        htil = jnp.tanh(G[:, 2 * _H:]
                        + jnp.dot(r * Hst, wh2_ref[...],
                                  preferred_element_type=jnp.float32))
        Hst = z * Hst + (1.0 - z) * htil
        if full_out:
            out_ref[t] = a_ref[t] * Hst + xseq_ref[t]
        return Hst

    Hst = jax.lax.fori_loop(0, _T, step, Hst0)
    hl = a_ref[_T - 1] * Hst
    hlast_ref[...] = hl
    if not full_out:
        out_ref[...] = hl + xseq_ref[_T - 1]


def _a3tgcn_layer(agg, xseq, norm, hprev, Wx, Wh, b, att, n, blk, full_out):
    nb = n // blk
    wx_cat = jnp.concatenate([Wx[0], Wx[1], Wx[2]], axis=1)
    whzr = jnp.concatenate([Wh[0], Wh[1]], axis=1)
    b_cat = jnp.concatenate([b[0], b[1], b[2]])[None, :]
    a = jax.nn.softmax(att)
    a3 = jnp.broadcast_to(a[:, None, None], (_T, 1, _H))
    if full_out:
        out_shape = jax.ShapeDtypeStruct((_T, n, _H), jnp.float32)
        out_spec = pl.BlockSpec((_T, blk, _H), lambda i: (0, i, 0))
    else:
        out_shape = jax.ShapeDtypeStruct((n, _H), jnp.float32)
        out_spec = pl.BlockSpec((blk, _H), lambda i: (i, 0))
    out, hlast = pl.pallas_call(
        functools.partial(_gru_body, full_out),
        grid=(nb,),
        in_specs=[
            pl.BlockSpec((_T, blk, _H), lambda i: (0, i, 0)),
            pl.BlockSpec((_T, blk, _H), lambda i: (0, i, 0)),
            pl.BlockSpec((blk, 1), lambda i: (i, 0)),
            pl.BlockSpec((blk, _H), lambda i: (i, 0)),
            pl.BlockSpec((_H, 3 * _H), lambda i: (0, 0)),
            pl.BlockSpec((_H, 2 * _H), lambda i: (0, 0)),
            pl.BlockSpec((_H, _H), lambda i: (0, 0)),
            pl.BlockSpec((1, 3 * _H), lambda i: (0, 0)),
            pl.BlockSpec((_T, 1, _H), lambda i: (0, 0, 0)),
        ],
        out_specs=[out_spec, pl.BlockSpec((blk, _H), lambda i: (i, 0))],
        out_shape=[out_shape, jax.ShapeDtypeStruct((n, _H), jnp.float32)],
    )(agg, xseq, norm, hprev, wx_cat, whzr, Wh[2], b_cat, a3)
    return out, hlast


def _dec_body(v_ref, w_ref, b_ref, o_ref):
    o_ref[...] = _leaky(jnp.dot(v_ref[...], w_ref[...],
                                preferred_element_type=jnp.float32)
                        + b_ref[...])


def _decode(v, W, b):
    n = v.shape[0]
    return pl.pallas_call(
        _dec_body,
        in_specs=[pl.BlockSpec((n, _H), lambda: (0, 0)),
                  pl.BlockSpec((_H, _H), lambda: (0, 0)),
                  pl.BlockSpec((1, _H), lambda: (0, 0))],
        out_specs=pl.BlockSpec((n, _H), lambda: (0, 0)),
        out_shape=jax.ShapeDtypeStruct((n, _H), jnp.float32),
    )(v, W, b[None, :])


def _readout_body(ef_ref, m_ref, l_ref, xl_ref, w1_ref, b1_ref, w2_ref,
                  b2_ref, ws_ref, o_ref):
    xs = jnp.concatenate([ef_ref[...], m_ref[...], l_ref[...]], axis=1)
    h = _leaky(jnp.dot(xs, w1_ref[...], preferred_element_type=jnp.float32)
               + b1_ref[...])
    o_ref[...] = (jnp.dot(h, w2_ref[...], preferred_element_type=jnp.float32)
                  + b2_ref[...]
                  + jnp.dot(xs, ws_ref[...], preferred_element_type=jnp.float32)
                  + 0.3 * xl_ref[...])


def _readout(ef, m2f, l2f, x_last, W1, b1, W2, b2, Ws):
    blk = 2000
    return pl.pallas_call(
        _readout_body,
        grid=(_N // blk,),
        in_specs=[
            pl.BlockSpec((blk, _H), lambda i: (i, 0)),
            pl.BlockSpec((blk, _H), lambda i: (i, 0)),
            pl.BlockSpec((blk, _H), lambda i: (i, 0)),
            pl.BlockSpec((blk, _F), lambda i: (i, 0)),
            pl.BlockSpec((3 * _H, _H), lambda i: (0, 0)),
            pl.BlockSpec((1, _H), lambda i: (0, 0)),
            pl.BlockSpec((_H, _F), lambda i: (0, 0)),
            pl.BlockSpec((1, _F), lambda i: (0, 0)),
            pl.BlockSpec((3 * _H, _F), lambda i: (0, 0)),
        ],
        out_specs=pl.BlockSpec((blk, _F), lambda i: (i, 0)),
        out_shape=jax.ShapeDtypeStruct((_N, _F), jnp.float32),
    )(ef, m2f, l2f, x_last, W1, b1[None, :], W2, b2[None, :], Ws)


def _agg_T(xseq, src, dst, n):
    outs = [jax.ops.segment_sum(jnp.take(xseq[t], src, axis=0), dst,
                                num_segments=n) for t in range(_T)]
    return jnp.stack(outs, axis=0)


def _deg_norm(dst, n):
    deg = jax.ops.segment_sum(jnp.ones_like(dst, dtype=jnp.float32), dst,
                              num_segments=n)
    return (1.0 / jnp.clip(deg, 1.0))[:, None]


def kernel(x, edge_index, enc_W1, enc_b1, enc_W2, enc_b2, enc_Ws, cm_Wn,
           cm_bn, cm_We, cm_be, cl_Wn, cl_bn, cl_We, cl_be, gru_Wx, gru_Wh,
           gru_b, att, dm_W, dm_b, dl_W, dl_b, ro_W1, ro_b1, ro_W2, ro_b2,
           ro_Ws):
    src, dst = edge_index[0], edge_index[1]
    msrc, mdst = src[::4] // 4, dst[::4] // 4
    lsrc, ldst = src[::16] // 16, dst[::16] // 16

    norm_f = _deg_norm(dst, _N)
    norm_m = _deg_norm(mdst, _NMP)
    norm_l = _deg_norm(ldst, _NLP)

    xT = jnp.transpose(x, (1, 0, 2))
    xTp = jnp.concatenate(
        [xT, jnp.zeros((_T, _NP - _N, _F), jnp.float32)], axis=1)
    enc = _encode(xT, enc_W1, enc_b1, enc_W2, enc_b2, enc_Ws)
    mid = _coarsen_mid(xTp, cm_Wn, cm_bn)
    large = _coarsen_large(xTp, cl_Wn, cl_bn)

    hf = jnp.zeros((_N, _H), jnp.float32)
    hm = jnp.zeros((_NMP, _H), jnp.float32)
    hl = jnp.zeros((_NLP, _H), jnp.float32)

    for layer in range(_L):
        full = layer < _L - 1
        agg_f = _agg_T(enc, src, dst, _N)
        agg_m = _agg_T(mid, msrc, mdst, _NMP)
        agg_l = _agg_T(large, lsrc, ldst, _NLP)
        enc, hf = _a3tgcn_layer(agg_f, enc, norm_f, hf, gru_Wx[0, layer],
                                gru_Wh[0, layer], gru_b[0, layer],
                                att[0, layer], _N, 400, full)
        mid, hm = _a3tgcn_layer(agg_m, mid, norm_m, hm, gru_Wx[1, layer],
                                gru_Wh[1, layer], gru_b[1, layer],
                                att[1, layer], _NMP, 512, full)
        large, hl = _a3tgcn_layer(agg_l, large, norm_l, hl, gru_Wx[2, layer],
                                  gru_Wh[2, layer], gru_b[2, layer],
                                  att[2, layer], _NLP, _NLP, full)

    m2f = _decode(mid, dm_W, dm_b)[:_NM]
    l2f = _decode(large, dl_W, dl_b)[:_NL]
    m2f = jnp.broadcast_to(m2f[:, None, :], (_NM, 4, _H)).reshape(_N, _H)
    l2f = jnp.broadcast_to(l2f[:, None, :], (_NL, 16, _H)).reshape(_N, _H)

    out = _readout(enc, m2f, l2f, x[:, -1, :], ro_W1, ro_b1, ro_W2, ro_b2,
                   ro_Ws)
    return out[:, None, :]


# speedup vs baseline: 3.3722x; 1.9327x over previous
import functools

import jax
import jax.numpy as jnp
from jax.experimental import pallas as pl

_N = 10000
_E = 160000
_F = 26
_H = 128
_T = 20
_L = 2
_NM = 2500
_NL = 625
_NMP = 2560
_NLP = 640
_NP = 10240


def _leaky(v):
    return jnp.where(v > 0, v, 0.01 * v)


def _enc_body(x_ref, w1_ref, b1_ref, w2_ref, b2_ref, ws_ref, o_ref):
    xt = x_ref[0]
    h = _leaky(jnp.dot(xt, w1_ref[...], preferred_element_type=jnp.float32)
               + b1_ref[...])
    o_ref[0] = (jnp.dot(h, w2_ref[...], preferred_element_type=jnp.float32)
                + b2_ref[...]
                + jnp.dot(xt, ws_ref[...], preferred_element_type=jnp.float32))


def _encode(xT, W1, b1, W2, b2, Ws):
    return pl.pallas_call(
        _enc_body,
        grid=(_T,),
        in_specs=[
            pl.BlockSpec((1, _N, _F), lambda t: (t, 0, 0)),
            pl.BlockSpec((_F, _H), lambda t: (0, 0)),
            pl.BlockSpec((1, _H), lambda t: (0, 0)),
            pl.BlockSpec((_H, _H), lambda t: (0, 0)),
            pl.BlockSpec((1, _H), lambda t: (0, 0)),
            pl.BlockSpec((_F, _H), lambda t: (0, 0)),
        ],
        out_specs=pl.BlockSpec((1, _N, _H), lambda t: (t, 0, 0)),
        out_shape=jax.ShapeDtypeStruct((_T, _N, _H), jnp.float32),
    )(xT, W1, b1[None, :], W2, b2[None, :], Ws)


def _coarsen_mid_body(x_ref, wn_ref, bn_ref, o_ref):
    xt = x_ref[0]
    h = _leaky(jnp.dot(xt, wn_ref[...], preferred_element_type=jnp.float32)
               + bn_ref[...])
    r = jax.lax.broadcasted_iota(jnp.int32, (512, 2048), 0)
    c = jax.lax.broadcasted_iota(jnp.int32, (512, 2048), 1)
    pool = jnp.where(c // 4 == r, 0.25, 0.0)
    o_ref[0] = jnp.dot(pool, h, preferred_element_type=jnp.float32)


def _coarsen_mid(xTp, Wn, bn):
    return pl.pallas_call(
        _coarsen_mid_body,
        grid=(_T, 5),
        in_specs=[
            pl.BlockSpec((1, 2048, _F), lambda t, i: (t, i, 0)),
            pl.BlockSpec((_F, _H), lambda t, i: (0, 0)),
            pl.BlockSpec((1, _H), lambda t, i: (0, 0)),
        ],
        out_specs=pl.BlockSpec((1, 512, _H), lambda t, i: (t, i, 0)),
        out_shape=jax.ShapeDtypeStruct((_T, _NMP, _H), jnp.float32),
    )(xTp, Wn, bn[None, :])


def _coarsen_large_body(x_ref, wn_ref, bn_ref, o_ref):
    xt = x_ref[0]
    h = _leaky(jnp.dot(xt, wn_ref[...], preferred_element_type=jnp.float32)
               + bn_ref[...])
    o_ref[0] = h.reshape(_NLP, 16, _H).sum(axis=1) * (1.0 / 16.0)


def _coarsen_large(xTp, Wn, bn):
    return pl.pallas_call(
        _coarsen_large_body,
        grid=(_T,),
        in_specs=[
            pl.BlockSpec((1, _NP, _F), lambda t: (t, 0, 0)),
            pl.BlockSpec((_F, _H), lambda t: (0, 0)),
            pl.BlockSpec((1, _H), lambda t: (0, 0)),
        ],
        out_specs=pl.BlockSpec((1, _NLP, _H), lambda t: (t, 0, 0)),
        out_shape=jax.ShapeDtypeStruct((_T, _NLP, _H), jnp.float32),
    )(xTp, Wn, bn[None, :])


def _gru_body(full_out, agg_ref, xseq_ref, norm_ref, hprev_ref, wx_ref,
              whzr_ref, wh2_ref, b_ref, a_ref, out_ref, hlast_ref):
    Hst0 = hprev_ref[...]

    def step(t, Hst):
        aggt = agg_ref[t]
        G = (norm_ref[...]
             * jnp.dot(aggt, wx_ref[...], preferred_element_type=jnp.float32)
             + b_ref[...])
        Hw = jnp.dot(Hst, whzr_ref[...], preferred_element_type=jnp.float32)
        z = jax.nn.sigmoid(G[:, :_H] + Hw[:, :_H])
        r = jax.nn.sigmoid(G[:, _H:2 * _H] + Hw[:, _H:])
        htil = jnp.tanh(G[:, 2 * _H:]
                        + jnp.dot(r * Hst, wh2_ref[...],
                                  preferred_element_type=jnp.float32))
        Hst = z * Hst + (1.0 - z) * htil
        if full_out:
            out_ref[t] = a_ref[t] * Hst + xseq_ref[t]
        return Hst

    Hst = jax.lax.fori_loop(0, _T, step, Hst0)
    hl = a_ref[_T - 1] * Hst
    hlast_ref[...] = hl
    if not full_out:
        out_ref[...] = hl + xseq_ref[_T - 1]


def _a3tgcn_layer(agg, xseq, norm, hprev, Wx, Wh, b, att, n, blk, full_out):
    nb = n // blk
    wx_cat = jnp.concatenate([Wx[0], Wx[1], Wx[2]], axis=1)
    whzr = jnp.concatenate([Wh[0], Wh[1]], axis=1)
    b_cat = jnp.concatenate([b[0], b[1], b[2]])[None, :]
    a = jax.nn.softmax(att)
    a3 = jnp.broadcast_to(a[:, None, None], (_T, 1, _H))
    if full_out:
        out_shape = jax.ShapeDtypeStruct((_T, n, _H), jnp.float32)
        out_spec = pl.BlockSpec((_T, blk, _H), lambda i: (0, i, 0))
    else:
        out_shape = jax.ShapeDtypeStruct((n, _H), jnp.float32)
        out_spec = pl.BlockSpec((blk, _H), lambda i: (i, 0))
    out, hlast = pl.pallas_call(
        functools.partial(_gru_body, full_out),
        grid=(nb,),
        in_specs=[
            pl.BlockSpec((_T, blk, _H), lambda i: (0, i, 0)),
            pl.BlockSpec((_T, blk, _H), lambda i: (0, i, 0)),
            pl.BlockSpec((blk, 1), lambda i: (i, 0)),
            pl.BlockSpec((blk, _H), lambda i: (i, 0)),
            pl.BlockSpec((_H, 3 * _H), lambda i: (0, 0)),
            pl.BlockSpec((_H, 2 * _H), lambda i: (0, 0)),
            pl.BlockSpec((_H, _H), lambda i: (0, 0)),
            pl.BlockSpec((1, 3 * _H), lambda i: (0, 0)),
            pl.BlockSpec((_T, 1, _H), lambda i: (0, 0, 0)),
        ],
        out_specs=[out_spec, pl.BlockSpec((blk, _H), lambda i: (i, 0))],
        out_shape=[out_shape, jax.ShapeDtypeStruct((n, _H), jnp.float32)],
    )(agg, xseq, norm, hprev, wx_cat, whzr, Wh[2], b_cat, a3)
    return out, hlast


def _dec_body(v_ref, w_ref, b_ref, o_ref):
    o_ref[...] = _leaky(jnp.dot(v_ref[...], w_ref[...],
                                preferred_element_type=jnp.float32)
                        + b_ref[...])


def _decode(v, W, b):
    n = v.shape[0]
    return pl.pallas_call(
        _dec_body,
        in_specs=[pl.BlockSpec((n, _H), lambda: (0, 0)),
                  pl.BlockSpec((_H, _H), lambda: (0, 0)),
                  pl.BlockSpec((1, _H), lambda: (0, 0))],
        out_specs=pl.BlockSpec((n, _H), lambda: (0, 0)),
        out_shape=jax.ShapeDtypeStruct((n, _H), jnp.float32),
    )(v, W, b[None, :])


def _readout_body(ef_ref, m_ref, l_ref, xl_ref, w1_ref, b1_ref, w2_ref,
                  b2_ref, ws_ref, o_ref):
    xs = jnp.concatenate([ef_ref[...], m_ref[...], l_ref[...]], axis=1)
    h = _leaky(jnp.dot(xs, w1_ref[...], preferred_element_type=jnp.float32)
               + b1_ref[...])
    o_ref[...] = (jnp.dot(h, w2_ref[...], preferred_element_type=jnp.float32)
                  + b2_ref[...]
                  + jnp.dot(xs, ws_ref[...], preferred_element_type=jnp.float32)
                  + 0.3 * xl_ref[...])


def _readout(ef, m2f, l2f, x_last, W1, b1, W2, b2, Ws):
    blk = 2000
    return pl.pallas_call(
        _readout_body,
        grid=(_N // blk,),
        in_specs=[
            pl.BlockSpec((blk, _H), lambda i: (i, 0)),
            pl.BlockSpec((blk, _H), lambda i: (i, 0)),
            pl.BlockSpec((blk, _H), lambda i: (i, 0)),
            pl.BlockSpec((blk, _F), lambda i: (i, 0)),
            pl.BlockSpec((3 * _H, _H), lambda i: (0, 0)),
            pl.BlockSpec((1, _H), lambda i: (0, 0)),
            pl.BlockSpec((_H, _F), lambda i: (0, 0)),
            pl.BlockSpec((1, _F), lambda i: (0, 0)),
            pl.BlockSpec((3 * _H, _F), lambda i: (0, 0)),
        ],
        out_specs=pl.BlockSpec((blk, _F), lambda i: (i, 0)),
        out_shape=jax.ShapeDtypeStruct((_N, _F), jnp.float32),
    )(ef, m2f, l2f, x_last, W1, b1[None, :], W2, b2[None, :], Ws)


def _agg_T(xseq, src, dst, n):
    xn = jnp.transpose(xseq, (1, 0, 2)).reshape(n, _T * _H)
    agg = jax.ops.segment_sum(jnp.take(xn, src, axis=0), dst, num_segments=n)
    return jnp.transpose(agg.reshape(n, _T, _H), (1, 0, 2))


def _deg_norm(dst, n):
    deg = jax.ops.segment_sum(jnp.ones_like(dst, dtype=jnp.float32), dst,
                              num_segments=n)
    return (1.0 / jnp.clip(deg, 1.0))[:, None]


def kernel(x, edge_index, enc_W1, enc_b1, enc_W2, enc_b2, enc_Ws, cm_Wn,
           cm_bn, cm_We, cm_be, cl_Wn, cl_bn, cl_We, cl_be, gru_Wx, gru_Wh,
           gru_b, att, dm_W, dm_b, dl_W, dl_b, ro_W1, ro_b1, ro_W2, ro_b2,
           ro_Ws):
    src, dst = edge_index[0], edge_index[1]
    msrc, mdst = src[::4] // 4, dst[::4] // 4
    lsrc, ldst = src[::16] // 16, dst[::16] // 16

    norm_f = _deg_norm(dst, _N)
    norm_m = _deg_norm(mdst, _NMP)
    norm_l = _deg_norm(ldst, _NLP)

    xT = jnp.transpose(x, (1, 0, 2))
    xTp = jnp.concatenate(
        [xT, jnp.zeros((_T, _NP - _N, _F), jnp.float32)], axis=1)
    enc = _encode(xT, enc_W1, enc_b1, enc_W2, enc_b2, enc_Ws)
    mid = _coarsen_mid(xTp, cm_Wn, cm_bn)
    large = _coarsen_large(xTp, cl_Wn, cl_bn)

    hf = jnp.zeros((_N, _H), jnp.float32)
    hm = jnp.zeros((_NMP, _H), jnp.float32)
    hl = jnp.zeros((_NLP, _H), jnp.float32)

    for layer in range(_L):
        full = layer < _L - 1
        agg_f = _agg_T(enc, src, dst, _N)
        agg_m = _agg_T(mid, msrc, mdst, _NMP)
        agg_l = _agg_T(large, lsrc, ldst, _NLP)
        enc, hf = _a3tgcn_layer(agg_f, enc, norm_f, hf, gru_Wx[0, layer],
                                gru_Wh[0, layer], gru_b[0, layer],
                                att[0, layer], _N, 400, full)
        mid, hm = _a3tgcn_layer(agg_m, mid, norm_m, hm, gru_Wx[1, layer],
                                gru_Wh[1, layer], gru_b[1, layer],
                                att[1, layer], _NMP, 512, full)
        large, hl = _a3tgcn_layer(agg_l, large, norm_l, hl, gru_Wx[2, layer],
                                  gru_Wh[2, layer], gru_b[2, layer],
                                  att[2, layer], _NLP, _NLP, full)

    m2f = _decode(mid, dm_W, dm_b)[:_NM]
    l2f = _decode(large, dl_W, dl_b)[:_NL]
    m2f = jnp.broadcast_to(m2f[:, None, :], (_NM, 4, _H)).reshape(_N, _H)
    l2f = jnp.broadcast_to(l2f[:, None, :], (_NL, 16, _H)).reshape(_N, _H)

    out = _readout(enc, m2f, l2f, x[:, -1, :], ro_W1, ro_b1, ro_W2, ro_b2,
                   ro_Ws)
    return out[:, None, :]


# speedup vs baseline: 3.4086x; 1.0108x over previous
import functools

import jax
import jax.numpy as jnp
from jax.experimental import pallas as pl

_N = 10000
_E = 160000
_F = 26
_H = 128
_T = 20
_L = 2
_NM = 2500
_NL = 625
_NMP = 2560
_NLP = 640
_NP = 10240


def _leaky(v):
    return jnp.where(v > 0, v, 0.01 * v)


def _enc_body(x_ref, w1_ref, b1_ref, w2_ref, b2_ref, ws_ref, o_ref):
    xt = x_ref[0]
    h = _leaky(jnp.dot(xt, w1_ref[...], preferred_element_type=jnp.float32)
               + b1_ref[...])
    o_ref[0] = (jnp.dot(h, w2_ref[...], preferred_element_type=jnp.float32)
                + b2_ref[...]
                + jnp.dot(xt, ws_ref[...], preferred_element_type=jnp.float32))


def _encode(xT, W1, b1, W2, b2, Ws):
    return pl.pallas_call(
        _enc_body,
        grid=(_T,),
        in_specs=[
            pl.BlockSpec((1, _N, _F), lambda t: (t, 0, 0)),
            pl.BlockSpec((_F, _H), lambda t: (0, 0)),
            pl.BlockSpec((1, _H), lambda t: (0, 0)),
            pl.BlockSpec((_H, _H), lambda t: (0, 0)),
            pl.BlockSpec((1, _H), lambda t: (0, 0)),
            pl.BlockSpec((_F, _H), lambda t: (0, 0)),
        ],
        out_specs=pl.BlockSpec((1, _N, _H), lambda t: (t, 0, 0)),
        out_shape=jax.ShapeDtypeStruct((_T, _N, _H), jnp.float32),
    )(xT, W1, b1[None, :], W2, b2[None, :], Ws)


def _coarsen_mid_body(x_ref, wn_ref, bn_ref, o_ref):
    xt = x_ref[0]
    h = _leaky(jnp.dot(xt, wn_ref[...], preferred_element_type=jnp.float32)
               + bn_ref[...])
    r = jax.lax.broadcasted_iota(jnp.int32, (512, 2048), 0)
    c = jax.lax.broadcasted_iota(jnp.int32, (512, 2048), 1)
    pool = jnp.where(c // 4 == r, 0.25, 0.0)
    o_ref[0] = jnp.dot(pool, h, preferred_element_type=jnp.float32)


def _coarsen_mid(xTp, Wn, bn):
    return pl.pallas_call(
        _coarsen_mid_body,
        grid=(_T, 5),
        in_specs=[
            pl.BlockSpec((1, 2048, _F), lambda t, i: (t, i, 0)),
            pl.BlockSpec((_F, _H), lambda t, i: (0, 0)),
            pl.BlockSpec((1, _H), lambda t, i: (0, 0)),
        ],
        out_specs=pl.BlockSpec((1, 512, _H), lambda t, i: (t, i, 0)),
        out_shape=jax.ShapeDtypeStruct((_T, _NMP, _H), jnp.float32),
    )(xTp, Wn, bn[None, :])


def _coarsen_large_body(x_ref, wn_ref, bn_ref, o_ref):
    xt = x_ref[0]
    h = _leaky(jnp.dot(xt, wn_ref[...], preferred_element_type=jnp.float32)
               + bn_ref[...])
    o_ref[0] = h.reshape(_NLP, 16, _H).sum(axis=1) * (1.0 / 16.0)


def _coarsen_large(xTp, Wn, bn):
    return pl.pallas_call(
        _coarsen_large_body,
        grid=(_T,),
        in_specs=[
            pl.BlockSpec((1, _NP, _F), lambda t: (t, 0, 0)),
            pl.BlockSpec((_F, _H), lambda t: (0, 0)),
            pl.BlockSpec((1, _H), lambda t: (0, 0)),
        ],
        out_specs=pl.BlockSpec((1, _NLP, _H), lambda t: (t, 0, 0)),
        out_shape=jax.ShapeDtypeStruct((_T, _NLP, _H), jnp.float32),
    )(xTp, Wn, bn[None, :])


def _gru_body(full_out, agg_ref, xseq_ref, norm_ref, hprev_ref, wx_ref,
              whzr_ref, wh2_ref, b_ref, a_ref, out_ref, hlast_ref):
    Hst0 = hprev_ref[...]

    def step(t, Hst):
        aggt = agg_ref[t]
        G = (norm_ref[...]
             * jnp.dot(aggt, wx_ref[...], preferred_element_type=jnp.float32)
             + b_ref[...])
        Hw = jnp.dot(Hst, whzr_ref[...], preferred_element_type=jnp.float32)
        z = jax.nn.sigmoid(G[:, :_H] + Hw[:, :_H])
        r = jax.nn.sigmoid(G[:, _H:2 * _H] + Hw[:, _H:])
        htil = jnp.tanh(G[:, 2 * _H:]
                        + jnp.dot(r * Hst, wh2_ref[...],
                                  preferred_element_type=jnp.float32))
        Hst = z * Hst + (1.0 - z) * htil
        if full_out:
            out_ref[t] = a_ref[t] * Hst + xseq_ref[t]
        return Hst

    Hst = Hst0
    for t in range(_T):
        Hst = step(t, Hst)
    hl = a_ref[_T - 1] * Hst
    hlast_ref[...] = hl
    if not full_out:
        out_ref[...] = hl + xseq_ref[_T - 1]


def _a3tgcn_layer(agg, xseq, norm, hprev, Wx, Wh, b, att, n, blk, full_out):
    nb = n // blk
    wx_cat = jnp.concatenate([Wx[0], Wx[1], Wx[2]], axis=1)
    whzr = jnp.concatenate([Wh[0], Wh[1]], axis=1)
    b_cat = jnp.concatenate([b[0], b[1], b[2]])[None, :]
    a = jax.nn.softmax(att)
    a3 = jnp.broadcast_to(a[:, None, None], (_T, 1, _H))
    if full_out:
        out_shape = jax.ShapeDtypeStruct((_T, n, _H), jnp.float32)
        out_spec = pl.BlockSpec((_T, blk, _H), lambda i: (0, i, 0))
    else:
        out_shape = jax.ShapeDtypeStruct((n, _H), jnp.float32)
        out_spec = pl.BlockSpec((blk, _H), lambda i: (i, 0))
    out, hlast = pl.pallas_call(
        functools.partial(_gru_body, full_out),
        grid=(nb,),
        in_specs=[
            pl.BlockSpec((_T, blk, _H), lambda i: (0, i, 0)),
            pl.BlockSpec((_T, blk, _H), lambda i: (0, i, 0)),
            pl.BlockSpec((blk, 1), lambda i: (i, 0)),
            pl.BlockSpec((blk, _H), lambda i: (i, 0)),
            pl.BlockSpec((_H, 3 * _H), lambda i: (0, 0)),
            pl.BlockSpec((_H, 2 * _H), lambda i: (0, 0)),
            pl.BlockSpec((_H, _H), lambda i: (0, 0)),
            pl.BlockSpec((1, 3 * _H), lambda i: (0, 0)),
            pl.BlockSpec((_T, 1, _H), lambda i: (0, 0, 0)),
        ],
        out_specs=[out_spec, pl.BlockSpec((blk, _H), lambda i: (i, 0))],
        out_shape=[out_shape, jax.ShapeDtypeStruct((n, _H), jnp.float32)],
    )(agg, xseq, norm, hprev, wx_cat, whzr, Wh[2], b_cat, a3)
    return out, hlast


def _dec_body(v_ref, w_ref, b_ref, o_ref):
    o_ref[...] = _leaky(jnp.dot(v_ref[...], w_ref[...],
                                preferred_element_type=jnp.float32)
                        + b_ref[...])


def _decode(v, W, b):
    n = v.shape[0]
    return pl.pallas_call(
        _dec_body,
        in_specs=[pl.BlockSpec((n, _H), lambda: (0, 0)),
                  pl.BlockSpec((_H, _H), lambda: (0, 0)),
                  pl.BlockSpec((1, _H), lambda: (0, 0))],
        out_specs=pl.BlockSpec((n, _H), lambda: (0, 0)),
        out_shape=jax.ShapeDtypeStruct((n, _H), jnp.float32),
    )(v, W, b[None, :])


def _readout_body(ef_ref, m_ref, l_ref, xl_ref, w1_ref, b1_ref, w2_ref,
                  b2_ref, ws_ref, o_ref):
    xs = jnp.concatenate([ef_ref[...], m_ref[...], l_ref[...]], axis=1)
    h = _leaky(jnp.dot(xs, w1_ref[...], preferred_element_type=jnp.float32)
               + b1_ref[...])
    o_ref[...] = (jnp.dot(h, w2_ref[...], preferred_element_type=jnp.float32)
                  + b2_ref[...]
                  + jnp.dot(xs, ws_ref[...], preferred_element_type=jnp.float32)
                  + 0.3 * xl_ref[...])


def _readout(ef, m2f, l2f, x_last, W1, b1, W2, b2, Ws):
    blk = 2000
    return pl.pallas_call(
        _readout_body,
        grid=(_N // blk,),
        in_specs=[
            pl.BlockSpec((blk, _H), lambda i: (i, 0)),
            pl.BlockSpec((blk, _H), lambda i: (i, 0)),
            pl.BlockSpec((blk, _H), lambda i: (i, 0)),
            pl.BlockSpec((blk, _F), lambda i: (i, 0)),
            pl.BlockSpec((3 * _H, _H), lambda i: (0, 0)),
            pl.BlockSpec((1, _H), lambda i: (0, 0)),
            pl.BlockSpec((_H, _F), lambda i: (0, 0)),
            pl.BlockSpec((1, _F), lambda i: (0, 0)),
            pl.BlockSpec((3 * _H, _F), lambda i: (0, 0)),
        ],
        out_specs=pl.BlockSpec((blk, _F), lambda i: (i, 0)),
        out_shape=jax.ShapeDtypeStruct((_N, _F), jnp.float32),
    )(ef, m2f, l2f, x_last, W1, b1[None, :], W2, b2[None, :], Ws)


def _agg_T(xseq, src, dst, n):
    xn = jnp.transpose(xseq, (1, 0, 2)).reshape(n, _T * _H)
    agg = jax.ops.segment_sum(jnp.take(xn, src, axis=0), dst, num_segments=n)
    return jnp.transpose(agg.reshape(n, _T, _H), (1, 0, 2))


def _deg_norm(dst, n):
    deg = jax.ops.segment_sum(jnp.ones_like(dst, dtype=jnp.float32), dst,
                              num_segments=n)
    return (1.0 / jnp.clip(deg, 1.0))[:, None]


def kernel(x, edge_index, enc_W1, enc_b1, enc_W2, enc_b2, enc_Ws, cm_Wn,
           cm_bn, cm_We, cm_be, cl_Wn, cl_bn, cl_We, cl_be, gru_Wx, gru_Wh,
           gru_b, att, dm_W, dm_b, dl_W, dl_b, ro_W1, ro_b1, ro_W2, ro_b2,
           ro_Ws):
    src, dst = edge_index[0], edge_index[1]
    msrc, mdst = src[::4] // 4, dst[::4] // 4
    lsrc, ldst = src[::16] // 16, dst[::16] // 16

    norm_f = _deg_norm(dst, _N)
    norm_m = _deg_norm(mdst, _NMP)
    norm_l = _deg_norm(ldst, _NLP)

    xT = jnp.transpose(x, (1, 0, 2))
    xTp = jnp.concatenate(
        [xT, jnp.zeros((_T, _NP - _N, _F), jnp.float32)], axis=1)
    enc = _encode(xT, enc_W1, enc_b1, enc_W2, enc_b2, enc_Ws)
    mid = _coarsen_mid(xTp, cm_Wn, cm_bn)
    large = _coarsen_large(xTp, cl_Wn, cl_bn)

    hf = jnp.zeros((_N, _H), jnp.float32)
    hm = jnp.zeros((_NMP, _H), jnp.float32)
    hl = jnp.zeros((_NLP, _H), jnp.float32)

    for layer in range(_L):
        full = layer < _L - 1
        agg_f = _agg_T(enc, src, dst, _N)
        agg_m = _agg_T(mid, msrc, mdst, _NMP)
        agg_l = _agg_T(large, lsrc, ldst, _NLP)
        enc, hf = _a3tgcn_layer(agg_f, enc, norm_f, hf, gru_Wx[0, layer],
                                gru_Wh[0, layer], gru_b[0, layer],
                                att[0, layer], _N, 400, full)
        mid, hm = _a3tgcn_layer(agg_m, mid, norm_m, hm, gru_Wx[1, layer],
                                gru_Wh[1, layer], gru_b[1, layer],
                                att[1, layer], _NMP, 512, full)
        large, hl = _a3tgcn_layer(agg_l, large, norm_l, hl, gru_Wx[2, layer],
                                  gru_Wh[2, layer], gru_b[2, layer],
                                  att[2, layer], _NLP, _NLP, full)

    m2f = _decode(mid, dm_W, dm_b)[:_NM]
    l2f = _decode(large, dl_W, dl_b)[:_NL]
    m2f = jnp.broadcast_to(m2f[:, None, :], (_NM, 4, _H)).reshape(_N, _H)
    l2f = jnp.broadcast_to(l2f[:, None, :], (_NL, 16, _H)).reshape(_N, _H)

    out = _readout(enc, m2f, l2f, x[:, -1, :], ro_W1, ro_b1, ro_W2, ro_b2,
                   ro_Ws)
    return out[:, None, :]
